# baseline (speedup 1.0000x reference)
import functools
import math

import jax
import jax.numpy as jnp
from jax import lax
from jax.experimental import pallas as pl
from jax.experimental.pallas import tpu as pltpu
from jax.experimental.pallas import tpu_sc as plsc

_D = 64
_L = 200
_B = 4096
_TOTAL = _B * _L
_NC = 2
_NS = 16
_NW = _NC * _NS
_NCALLS = 4
_CROWS = _TOTAL // _NCALLS
_ROWS_PER_W = _CROWS // _NW
_CHUNK = 128
_STEPS = _ROWS_PER_W // _CHUNK
_GW = 128
_SUB = _CHUNK // _GW
_LANES = 16
_SCALE = 2.0 * math.sqrt(float(_D))


def _make_pe(n, d):
    position = jnp.arange(0, n, dtype=jnp.float32)[:, None]
    div_term = jnp.exp(-jnp.arange(0, d, 2, dtype=jnp.float32) * math.log(10000.0) / d)
    pe = jnp.zeros((n, d), dtype=jnp.float32)
    pe = pe.at[:, 0::2].set(jnp.sin(position * div_term))
    pe = pe.at[:, 1::2].set(jnp.cos(position * div_term))
    return pe


def _sc_lookup(token_table, comb, idx, seg):
    mesh = plsc.VectorSubcoreMesh(core_axis_name="c", subcore_axis_name="s")

    vm = pltpu.VMEM
    @functools.partial(
        pl.kernel,
        out_type=jax.ShapeDtypeStruct((_CROWS * _D,), jnp.float32),
        mesh=mesh,
        compiler_params=pltpu.CompilerParams(use_tc_tiling_on_sc=False),
        scratch_types=[
            vm((_CHUNK,), jnp.int32), vm((_CHUNK,), jnp.int32),
            vm((_CHUNK,), jnp.int32), vm((_CHUNK,), jnp.int32),
            vm((_CHUNK,), jnp.int32), vm((_CHUNK,), jnp.int32),
            vm((_CHUNK, _D), jnp.float32), vm((_CHUNK, _D), jnp.float32),
            vm((_CHUNK, _D), jnp.float32), vm((_CHUNK, _D), jnp.float32),
            vm((_CHUNK * _D,), jnp.float32), vm((_CHUNK * _D,), jnp.float32),
            pltpu.SemaphoreType.DMA, pltpu.SemaphoreType.DMA,
            pltpu.SemaphoreType.DMA, pltpu.SemaphoreType.DMA,
        ],
    )
    def k(tok_hbm, comb_hbm, idx_hbm, seg_hbm, out_hbm,
          idx0, idx1, seg0, seg1, cidx0, cidx1,
          rows0, rows1, add0, add1, out0, out1, sg0, sg1, sw0, sw1):
        wid = lax.axis_index("s") * _NC + lax.axis_index("c")
        w_base = wid * _ROWS_PER_W
        idx_v = (idx0, idx1)
        seg_v = (seg0, seg1)
        cidx_v = (cidx0, cidx1)
        rows_v = (rows0, rows1)
        add_v = (add0, add1)
        out_v = (out0, out1)
        sg = (sg0, sg1)
        sw = (sw0, sw1)
        iota = lax.broadcasted_iota(jnp.int32, (_LANES,), 0)

        def prep_and_fire(k_step, p):
            base = w_base + k_step * _CHUNK
            pltpu.sync_copy(idx_hbm.at[pl.ds(base, _CHUNK)], idx_v[p])
            pltpu.sync_copy(seg_hbm.at[pl.ds(base, _CHUNK)], seg_v[p])

            @pl.loop(0, _CHUNK, step=_LANES)
            def _cidx(c0):
                pos = base + c0 + iota
                cidx_v[p][pl.ds(c0, _LANES)] = (
                    seg_v[p][pl.ds(c0, _LANES)] * _L + lax.rem(pos, _L))

            for j in range(_SUB):
                pltpu.async_copy(
                    tok_hbm.at[idx_v[p].at[pl.ds(j * _GW, _GW)]],
                    rows_v[p].at[pl.ds(j * _GW, _GW)], sg[p])
                pltpu.async_copy(
                    comb_hbm.at[cidx_v[p].at[pl.ds(j * _GW, _GW)]],
                    add_v[p].at[pl.ds(j * _GW, _GW)], sg[p])

        def wait_gathers(p):
            for j in range(_SUB):
                pltpu.make_async_copy(
                    tok_hbm.at[idx_v[p].at[pl.ds(j * _GW, _GW)]],
                    rows_v[p].at[pl.ds(j * _GW, _GW)], sg[p]).wait()
                pltpu.make_async_copy(
                    comb_hbm.at[cidx_v[p].at[pl.ds(j * _GW, _GW)]],
                    add_v[p].at[pl.ds(j * _GW, _GW)], sg[p]).wait()

        def wb_descr(k_step, p):
            base = w_base + k_step * _CHUNK
            return pltpu.make_async_copy(
                out_v[p], out_hbm.at[pl.ds(base * _D, _CHUNK * _D)], sw[p])

        prep_and_fire(0, 0)

        @pl.loop(0, _STEPS // 2)
        def _pipe(i):
            for p in range(2):
                k_step = 2 * i + p
                nxt = k_step + 1

                @pl.when(k_step >= 1)
                def _():
                    wb_descr(k_step - 1, 1 - p).wait()

                @pl.when(nxt < _STEPS)
                def _():
                    prep_and_fire(nxt, 1 - p)

                wait_gathers(p)

                @pl.loop(0, _CHUNK)
                def _fma(r):
                    for c0 in range(0, _D, _LANES):
                        slc = (pl.ds(r, 1), pl.ds(c0, _LANES))
                        out_v[p][pl.ds(r * _D + c0, _LANES)] = (
                            rows_v[p][slc] * _SCALE + add_v[p][slc]).reshape(_LANES)

                wb_descr(k_step, p).start()

        wb_descr(_STEPS - 1, (_STEPS - 1) % 2).wait()

    return k(token_table, comb, idx, seg)


def kernel(bert_inputs, segment_labels, token_table, segment_table):
    pe = _make_pe(_L, _D)
    comb = (segment_table[:, None, :].astype(jnp.float32)
            + pe[None, :, :]).reshape(3 * _L, _D)
    idx = bert_inputs.reshape(_TOTAL).astype(jnp.int32)
    seg = segment_labels.reshape(_TOTAL).astype(jnp.int32)
    tbl = token_table.astype(jnp.float32)
    outs = [
        _sc_lookup(tbl, comb,
                   lax.slice(idx, (c * _CROWS,), ((c + 1) * _CROWS,)),
                   lax.slice(seg, (c * _CROWS,), ((c + 1) * _CROWS,)))
        .reshape(_B // _NCALLS, _L, _D)
        for c in range(_NCALLS)
    ]
    return jnp.concatenate(outs, axis=0)


# speedup vs baseline: 1.2194x; 1.2194x over previous
import functools
import math

import jax
import jax.numpy as jnp
from jax import lax
from jax.experimental import pallas as pl
from jax.experimental.pallas import tpu as pltpu
from jax.experimental.pallas import tpu_sc as plsc

_D = 64
_L = 200
_B = 4096
_TOTAL = _B * _L
_NC = 2
_NS = 16
_NW = _NC * _NS
_NCALLS = 1
_CROWS = _TOTAL // _NCALLS
_ROWS_PER_W = _CROWS // _NW
_CHUNK = 256
_STEPS = _ROWS_PER_W // _CHUNK
_GW = 128
_SUB = _CHUNK // _GW
_LANES = 16
_SCALE = 2.0 * math.sqrt(float(_D))


def _make_pe(n, d):
    position = jnp.arange(0, n, dtype=jnp.float32)[:, None]
    div_term = jnp.exp(-jnp.arange(0, d, 2, dtype=jnp.float32) * math.log(10000.0) / d)
    pe = jnp.zeros((n, d), dtype=jnp.float32)
    pe = pe.at[:, 0::2].set(jnp.sin(position * div_term))
    pe = pe.at[:, 1::2].set(jnp.cos(position * div_term))
    return pe


def _sc_lookup(token_table, comb, idx, seg):
    mesh = plsc.VectorSubcoreMesh(core_axis_name="c", subcore_axis_name="s")

    vm = pltpu.VMEM
    @functools.partial(
        pl.kernel,
        out_type=jax.ShapeDtypeStruct((_CROWS * _D,), jnp.float32),
        mesh=mesh,
        compiler_params=pltpu.CompilerParams(use_tc_tiling_on_sc=False),
        scratch_types=[
            vm((_CHUNK,), jnp.int32), vm((_CHUNK,), jnp.int32),
            vm((_CHUNK,), jnp.int32), vm((_CHUNK,), jnp.int32),
            vm((_CHUNK,), jnp.int32), vm((_CHUNK,), jnp.int32),
            vm((_CHUNK, _D), jnp.float32), vm((_CHUNK, _D), jnp.float32),
            vm((_CHUNK, _D), jnp.float32), vm((_CHUNK, _D), jnp.float32),
            vm((_CHUNK * _D,), jnp.float32), vm((_CHUNK * _D,), jnp.float32),
            pltpu.SemaphoreType.DMA, pltpu.SemaphoreType.DMA,
            pltpu.SemaphoreType.DMA, pltpu.SemaphoreType.DMA,
        ],
    )
    def k(tok_hbm, comb_hbm, idx_hbm, seg_hbm, out_hbm,
          idx0, idx1, seg0, seg1, cidx0, cidx1,
          rows0, rows1, add0, add1, out0, out1, sg0, sg1, sw0, sw1):
        wid = lax.axis_index("s") * _NC + lax.axis_index("c")
        w_base = wid * _ROWS_PER_W
        idx_v = (idx0, idx1)
        seg_v = (seg0, seg1)
        cidx_v = (cidx0, cidx1)
        rows_v = (rows0, rows1)
        add_v = (add0, add1)
        out_v = (out0, out1)
        sg = (sg0, sg1)
        sw = (sw0, sw1)
        iota = lax.broadcasted_iota(jnp.int32, (_LANES,), 0)

        def prep_and_fire(k_step, p):
            base = w_base + k_step * _CHUNK
            pltpu.sync_copy(idx_hbm.at[pl.ds(base, _CHUNK)], idx_v[p])
            pltpu.sync_copy(seg_hbm.at[pl.ds(base, _CHUNK)], seg_v[p])

            @pl.loop(0, _CHUNK, step=_LANES)
            def _cidx(c0):
                pos = base + c0 + iota
                cidx_v[p][pl.ds(c0, _LANES)] = (
                    seg_v[p][pl.ds(c0, _LANES)] * _L + lax.rem(pos, _L))

            for j in range(_SUB):
                pltpu.async_copy(
                    tok_hbm.at[idx_v[p].at[pl.ds(j * _GW, _GW)]],
                    rows_v[p].at[pl.ds(j * _GW, _GW)], sg[p])
                pltpu.async_copy(
                    comb_hbm.at[cidx_v[p].at[pl.ds(j * _GW, _GW)]],
                    add_v[p].at[pl.ds(j * _GW, _GW)], sg[p])

        def wait_gathers(p):
            for j in range(_SUB):
                pltpu.make_async_copy(
                    tok_hbm.at[idx_v[p].at[pl.ds(j * _GW, _GW)]],
                    rows_v[p].at[pl.ds(j * _GW, _GW)], sg[p]).wait()
                pltpu.make_async_copy(
                    comb_hbm.at[cidx_v[p].at[pl.ds(j * _GW, _GW)]],
                    add_v[p].at[pl.ds(j * _GW, _GW)], sg[p]).wait()

        def wb_descr(k_step, p):
            base = w_base + k_step * _CHUNK
            return pltpu.make_async_copy(
                out_v[p], out_hbm.at[pl.ds(base * _D, _CHUNK * _D)], sw[p])

        prep_and_fire(0, 0)

        @pl.loop(0, _STEPS // 2)
        def _pipe(i):
            for p in range(2):
                k_step = 2 * i + p
                nxt = k_step + 1

                @pl.when(k_step >= 1)
                def _():
                    wb_descr(k_step - 1, 1 - p).wait()

                @pl.when(nxt < _STEPS)
                def _():
                    prep_and_fire(nxt, 1 - p)

                wait_gathers(p)

                @pl.loop(0, _CHUNK, step=4)
                def _fma(r):
                    for dr in range(4):
                        for c0 in range(0, _D, _LANES):
                            slc = (pl.ds(r + dr, 1), pl.ds(c0, _LANES))
                            out_v[p][pl.ds((r + dr) * _D + c0, _LANES)] = (
                                rows_v[p][slc] * _SCALE
                                + add_v[p][slc]).reshape(_LANES)

                wb_descr(k_step, p).start()

        wb_descr(_STEPS - 1, (_STEPS - 1) % 2).wait()

    return k(token_table, comb, idx, seg)


def kernel(bert_inputs, segment_labels, token_table, segment_table):
    pe = _make_pe(_L, _D)
    comb = (segment_table[:, None, :].astype(jnp.float32)
            + pe[None, :, :]).reshape(3 * _L, _D)
    idx = bert_inputs.reshape(_TOTAL).astype(jnp.int32)
    seg = segment_labels.reshape(_TOTAL).astype(jnp.int32)
    tbl = token_table.astype(jnp.float32)
    out = _sc_lookup(tbl, comb, idx, seg)
    return out.reshape(_B, _L, _D)
